# baseline (speedup 1.0000x reference)
import dataclasses
import functools

import jax
import jax.numpy as jnp
from jax import lax
from jax.experimental import pallas as pl
from jax.experimental.pallas import tpu as pltpu
from jax.experimental.pallas import tpu_sc as plsc

ALPHA = 0.1
K_HOPS = 10

NC = 2
NS = 16
LANES = 16
EB = 128
SB = 40
CH = 104


def _sc_appnp(x2, src3, dst3, val3, zeros, n_nodes, dh, nb):
    rows_main = (n_nodes // NS) & ~7
    rem = n_nodes - rows_main * NS
    n_ch = rows_main // CH
    nsg = nb // SB
    assert n_ch * CH == rows_main and CH <= EB and rem <= EB
    assert nsg * SB == nb and nsg % 2 == 0

    mesh = plsc.VectorSubcoreMesh(core_axis_name="c", subcore_axis_name="s")

    cp = pltpu.CompilerParams()
    fields = pltpu.CompilerParams.__dataclass_fields__
    if "needs_layout_passes" in fields:
        cp = dataclasses.replace(cp, needs_layout_passes=False)
    if "use_tc_tiling_on_sc" in fields:
        cp = dataclasses.replace(cp, use_tc_tiling_on_sc=False)

    @functools.partial(
        pl.kernel,
        out_type=jax.ShapeDtypeStruct((NC * n_nodes, dh), jnp.float32),
        mesh=mesh,
        compiler_params=cp,
        scratch_types=[
            pltpu.VMEM((2, SB, EB), jnp.int32),
            pltpu.VMEM((2, SB, EB), jnp.int32),
            pltpu.VMEM((2, SB, EB), jnp.float32),
            pltpu.VMEM((2, EB, dh), jnp.float32),
            pltpu.VMEM_SHARED((n_nodes, dh), jnp.float32),
            pltpu.VMEM_SHARED((n_nodes, dh), jnp.float32),
            pltpu.SemaphoreType.DMA,
            pltpu.SemaphoreType.DMA,
            pltpu.SemaphoreType.DMA,
            pltpu.SemaphoreType.DMA,
            pltpu.SemaphoreType.DMA,
            pltpu.SemaphoreType.DMA,
        ],
    )
    def prop(x2_hbm, src_hbm, dst_hbm, val_hbm, zero_hbm, out_hbm,
             seg_src, seg_dst, seg_val, rows_v, h_sh, acc_sh,
             sem_s0, sem_s1, sem_d0, sem_d1, sem_g0, sem_g1):
        cid = lax.axis_index("c")
        sid = lax.axis_index("s")
        sem_s = (sem_s0, sem_s1)
        sem_d = (sem_d0, sem_d1)
        sem_g = (sem_g0, sem_g1)
        r0 = sid * rows_main

        def src_copy(s, sp):
            return pltpu.make_async_copy(
                src_hbm.at[sid, pl.ds(s * SB, SB)], seg_src.at[sp], sem_s[sp])

        def dst_copy(s, sp):
            return pltpu.make_async_copy(
                dst_hbm.at[sid, pl.ds(s * SB, SB)], seg_dst.at[sp], sem_d[sp])

        def val_copy(s, sp):
            return pltpu.make_async_copy(
                val_hbm.at[sid, pl.ds(s * SB, SB)], seg_val.at[sp], sem_d[sp])

        def stage_seg(s, sp):
            src_copy(s, sp).start()
            dst_copy(s, sp).start()
            val_copy(s, sp).start()

        pltpu.sync_copy(x2_hbm.at[pl.ds(cid * n_nodes + r0, rows_main)],
                        h_sh.at[pl.ds(r0, rows_main)])
        pltpu.sync_copy(zero_hbm.at[pl.ds(r0, rows_main)],
                        acc_sh.at[pl.ds(r0, rows_main)])
        if rem:
            @pl.when(sid == NS - 1)
            def _():
                t0 = rows_main * NS
                pltpu.sync_copy(x2_hbm.at[pl.ds(cid * n_nodes + t0, rem)],
                                h_sh.at[pl.ds(t0, rem)])
                pltpu.sync_copy(zero_hbm.at[pl.ds(t0, rem)],
                                acc_sh.at[pl.ds(t0, rem)])

        plsc.subcore_barrier()

        def substep(s, sp, j, p):
            q = 1 - p
            pltpu.make_async_copy(
                h_sh.at[seg_src.at[sp, j]], rows_v.at[p], sem_g[p]).wait()

            @pl.when(j + 1 < SB)
            def _():
                pltpu.async_copy(
                    h_sh.at[seg_src.at[sp, j + 1]], rows_v.at[q], sem_g[q])

            @pl.when(j + 1 == SB)
            def _():
                @pl.when(s + 1 < nsg)
                def _():
                    src_copy(s + 1, 1 - sp).wait()
                    pltpu.async_copy(
                        h_sh.at[seg_src.at[1 - sp, 0]], rows_v.at[q],
                        sem_g[q])

            @pl.loop(0, EB // 2)
            def _(rh):
                ra = 2 * rh
                rb = 2 * rh + 1
                va = plsc.load_gather(
                    seg_val, [jnp.full((LANES,), sp, dtype=jnp.int32),
                              jnp.full((LANES,), j, dtype=jnp.int32),
                              jnp.full((LANES,), ra, dtype=jnp.int32)])
                vb = plsc.load_gather(
                    seg_val, [jnp.full((LANES,), sp, dtype=jnp.int32),
                              jnp.full((LANES,), j, dtype=jnp.int32),
                              jnp.full((LANES,), rb, dtype=jnp.int32)])
                for c in range(dh // LANES):
                    sl = pl.ds(c * LANES, LANES)
                    rows_v[p, ra, sl] = rows_v[p, ra, sl] * va
                    rows_v[p, rb, sl] = rows_v[p, rb, sl] * vb

            pltpu.sync_copy(rows_v.at[p], acc_sh.at[seg_dst.at[sp, j]],
                            add=True)

        def seg_body(s, sp):
            dst_copy(s, sp).wait()
            val_copy(s, sp).wait()

            @pl.loop(0, SB // 2)
            def _(jh):
                substep(s, sp, 2 * jh, 0)
                substep(s, sp, 2 * jh + 1, 1)

            @pl.when(s + 2 < nsg)
            def _():
                stage_seg(s + 2, sp)

        def combine_rows(row0, nrows):
            a_v = rows_v.at[0, pl.ds(0, nrows)]
            x_v = rows_v.at[1, pl.ds(0, nrows)]
            pltpu.sync_copy(acc_sh.at[pl.ds(row0, nrows)], a_v)
            pltpu.sync_copy(x2_hbm.at[pl.ds(cid * n_nodes + row0, nrows)], x_v)
            pltpu.sync_copy(zero_hbm.at[pl.ds(row0, nrows)],
                            acc_sh.at[pl.ds(row0, nrows)])

            @pl.loop(0, nrows)
            def _(r):
                for c in range(dh // LANES):
                    sl = pl.ds(c * LANES, LANES)
                    rows_v[0, r, sl] = ((1.0 - ALPHA) * rows_v[0, r, sl]
                                        + ALPHA * rows_v[1, r, sl])

            pltpu.sync_copy(a_v, h_sh.at[pl.ds(row0, nrows)])

        @pl.loop(0, K_HOPS)
        def _(t):
            stage_seg(0, 0)
            stage_seg(1, 1)
            src_copy(0, 0).wait()
            pltpu.async_copy(h_sh.at[seg_src.at[0, 0]], rows_v.at[0], sem_g0)

            @pl.loop(0, nsg // 2)
            def _(i):
                seg_body(2 * i, 0)
                seg_body(2 * i + 1, 1)

            plsc.subcore_barrier()

            @pl.loop(0, n_ch)
            def _(j):
                combine_rows(r0 + j * CH, CH)

            if rem:
                @pl.when(sid == NS - 1)
                def _():
                    combine_rows(rows_main * NS, rem)

            plsc.subcore_barrier()

        pltpu.sync_copy(h_sh.at[pl.ds(r0, rows_main)],
                        out_hbm.at[pl.ds(cid * n_nodes + r0, rows_main)])
        if rem:
            @pl.when(sid == NS - 1)
            def _():
                t0 = rows_main * NS
                pltpu.sync_copy(
                    h_sh.at[pl.ds(t0, rem)],
                    out_hbm.at[pl.ds(cid * n_nodes + t0, rem)])

    return prop(x2, src3, dst3, val3, zeros)


def kernel(x, edge_index, adj_values):
    n_nodes, d = x.shape
    dh = d // NC
    dst = edge_index[0]
    src = edge_index[1]
    e = dst.shape[0]

    nb = -(-e // (NS * EB))
    nb = -(-nb // (2 * SB)) * (2 * SB)
    e_pad = nb * EB * NS
    pad = e_pad - e
    if pad:
        src = jnp.concatenate([src, jnp.zeros((pad,), src.dtype)])
        dst = jnp.concatenate([dst, jnp.zeros((pad,), dst.dtype)])
        adj = jnp.concatenate([adj_values, jnp.zeros((pad,), adj_values.dtype)])
    else:
        adj = adj_values
    src3 = src.reshape(NS, nb, EB)
    dst3 = dst.reshape(NS, nb, EB)
    val3 = adj.reshape(NS, nb, EB)
    zeros = jnp.zeros((n_nodes, dh), jnp.float32)

    x2 = jnp.concatenate([x[:, :dh], x[:, dh:]], axis=0)

    h2 = _sc_appnp(x2, src3, dst3, val3, zeros, n_nodes, dh, nb)

    return jnp.concatenate([h2[:n_nodes], h2[n_nodes:]], axis=1)


# speedup vs baseline: 1.0477x; 1.0477x over previous
import dataclasses
import functools

import jax
import jax.numpy as jnp
from jax import lax
from jax.experimental import pallas as pl
from jax.experimental.pallas import tpu as pltpu
from jax.experimental.pallas import tpu_sc as plsc

ALPHA = 0.1
K_HOPS = 10

NC = 2
NS = 16
LANES = 16
EB = 128
SB = 40
CH = 104


def _sc_appnp(x2, src3, dst3, val3, zeros, n_nodes, dh, nb):
    rows_main = (n_nodes // NS) & ~7
    rem = n_nodes - rows_main * NS
    n_ch = rows_main // CH
    nsg = nb // SB
    assert n_ch * CH == rows_main and CH <= EB and rem <= EB
    assert nsg * SB == nb and nsg % 2 == 0

    mesh = plsc.VectorSubcoreMesh(core_axis_name="c", subcore_axis_name="s")

    cp = pltpu.CompilerParams()
    fields = pltpu.CompilerParams.__dataclass_fields__
    if "needs_layout_passes" in fields:
        cp = dataclasses.replace(cp, needs_layout_passes=False)
    if "use_tc_tiling_on_sc" in fields:
        cp = dataclasses.replace(cp, use_tc_tiling_on_sc=False)

    @functools.partial(
        pl.kernel,
        out_type=jax.ShapeDtypeStruct((NC * n_nodes, dh), jnp.float32),
        mesh=mesh,
        compiler_params=cp,
        scratch_types=[
            pltpu.VMEM((2, SB, EB), jnp.int32),
            pltpu.VMEM((2, SB, EB), jnp.int32),
            pltpu.VMEM((2, SB, EB), jnp.float32),
            pltpu.VMEM((2, EB, dh), jnp.float32),
            pltpu.VMEM_SHARED((n_nodes, dh), jnp.float32),
            pltpu.VMEM_SHARED((n_nodes, dh), jnp.float32),
            pltpu.SemaphoreType.DMA,
            pltpu.SemaphoreType.DMA,
            pltpu.SemaphoreType.DMA,
            pltpu.SemaphoreType.DMA,
            pltpu.SemaphoreType.DMA,
            pltpu.SemaphoreType.DMA,
        ],
    )
    def prop(x2_hbm, src_hbm, dst_hbm, val_hbm, zero_hbm, out_hbm,
             seg_src, seg_dst, seg_val, rows_v, h_sh, acc_sh,
             sem_s0, sem_s1, sem_d0, sem_d1, sem_g0, sem_g1):
        cid = lax.axis_index("c")
        sid = lax.axis_index("s")
        sem_s = (sem_s0, sem_s1)
        sem_d = (sem_d0, sem_d1)
        sem_g = (sem_g0, sem_g1)
        r0 = sid * rows_main

        def src_copy(s, sp):
            return pltpu.make_async_copy(
                src_hbm.at[sid, pl.ds(s * SB, SB)], seg_src.at[sp], sem_s[sp])

        def dst_copy(s, sp):
            return pltpu.make_async_copy(
                dst_hbm.at[sid, pl.ds(s * SB, SB)], seg_dst.at[sp], sem_d[sp])

        def val_copy(s, sp):
            return pltpu.make_async_copy(
                val_hbm.at[sid, pl.ds(s * SB, SB)], seg_val.at[sp], sem_d[sp])

        def stage_seg(s, sp):
            src_copy(s, sp).start()
            dst_copy(s, sp).start()
            val_copy(s, sp).start()

        pltpu.sync_copy(x2_hbm.at[pl.ds(cid * n_nodes + r0, rows_main)],
                        h_sh.at[pl.ds(r0, rows_main)])
        pltpu.sync_copy(zero_hbm.at[pl.ds(r0, rows_main)],
                        acc_sh.at[pl.ds(r0, rows_main)])
        if rem:
            @pl.when(sid == NS - 1)
            def _():
                t0 = rows_main * NS
                pltpu.sync_copy(x2_hbm.at[pl.ds(cid * n_nodes + t0, rem)],
                                h_sh.at[pl.ds(t0, rem)])
                pltpu.sync_copy(zero_hbm.at[pl.ds(t0, rem)],
                                acc_sh.at[pl.ds(t0, rem)])

        plsc.subcore_barrier()

        def substep(s, sp, j, p):
            q = 1 - p
            pltpu.make_async_copy(
                h_sh.at[seg_src.at[sp, j]], rows_v.at[p], sem_g[p]).wait()

            @pl.when(j + 1 < SB)
            def _():
                pltpu.async_copy(
                    h_sh.at[seg_src.at[sp, j + 1]], rows_v.at[q], sem_g[q])

            @pl.when(j + 1 == SB)
            def _():
                @pl.when(s + 1 < nsg)
                def _():
                    src_copy(s + 1, 1 - sp).wait()
                    pltpu.async_copy(
                        h_sh.at[seg_src.at[1 - sp, 0]], rows_v.at[q],
                        sem_g[q])

            @pl.loop(0, EB)
            def _(r):
                vv = plsc.load_gather(
                    seg_val, [jnp.full((LANES,), sp, dtype=jnp.int32),
                              jnp.full((LANES,), j, dtype=jnp.int32),
                              jnp.full((LANES,), r, dtype=jnp.int32)])
                for c in range(dh // LANES):
                    sl = pl.ds(c * LANES, LANES)
                    rows_v[p, r, sl] = rows_v[p, r, sl] * vv

            pltpu.sync_copy(rows_v.at[p], acc_sh.at[seg_dst.at[sp, j]],
                            add=True)

        def seg_body(s, sp):
            dst_copy(s, sp).wait()
            val_copy(s, sp).wait()

            @pl.loop(0, SB // 2)
            def _(jh):
                substep(s, sp, 2 * jh, 0)
                substep(s, sp, 2 * jh + 1, 1)

            @pl.when(s + 2 < nsg)
            def _():
                stage_seg(s + 2, sp)

        def combine_rows(row0, nrows):
            a_v = rows_v.at[0, pl.ds(0, nrows)]
            x_v = rows_v.at[1, pl.ds(0, nrows)]
            pltpu.sync_copy(acc_sh.at[pl.ds(row0, nrows)], a_v)
            pltpu.sync_copy(x2_hbm.at[pl.ds(cid * n_nodes + row0, nrows)], x_v)
            pltpu.sync_copy(zero_hbm.at[pl.ds(row0, nrows)],
                            acc_sh.at[pl.ds(row0, nrows)])

            @pl.loop(0, nrows)
            def _(r):
                for c in range(dh // LANES):
                    sl = pl.ds(c * LANES, LANES)
                    rows_v[0, r, sl] = ((1.0 - ALPHA) * rows_v[0, r, sl]
                                        + ALPHA * rows_v[1, r, sl])

            pltpu.sync_copy(a_v, h_sh.at[pl.ds(row0, nrows)])

        @pl.loop(0, K_HOPS)
        def _(t):
            stage_seg(0, 0)
            stage_seg(1, 1)
            src_copy(0, 0).wait()
            pltpu.async_copy(h_sh.at[seg_src.at[0, 0]], rows_v.at[0], sem_g0)

            @pl.loop(0, nsg // 2)
            def _(i):
                seg_body(2 * i, 0)
                seg_body(2 * i + 1, 1)

            plsc.subcore_barrier()

            @pl.loop(0, n_ch)
            def _(j):
                combine_rows(r0 + j * CH, CH)

            if rem:
                @pl.when(sid == NS - 1)
                def _():
                    combine_rows(rows_main * NS, rem)

            plsc.subcore_barrier()

        pltpu.sync_copy(h_sh.at[pl.ds(r0, rows_main)],
                        out_hbm.at[pl.ds(cid * n_nodes + r0, rows_main)])
        if rem:
            @pl.when(sid == NS - 1)
            def _():
                t0 = rows_main * NS
                pltpu.sync_copy(
                    h_sh.at[pl.ds(t0, rem)],
                    out_hbm.at[pl.ds(cid * n_nodes + t0, rem)])

    return prop(x2, src3, dst3, val3, zeros)


def kernel(x, edge_index, adj_values):
    n_nodes, d = x.shape
    dh = d // NC
    dst = edge_index[0]
    src = edge_index[1]
    e = dst.shape[0]

    nb = -(-e // (NS * EB))
    nb = -(-nb // (2 * SB)) * (2 * SB)
    e_pad = nb * EB * NS
    pad = e_pad - e
    if pad:
        src = jnp.concatenate([src, jnp.zeros((pad,), src.dtype)])
        dst = jnp.concatenate([dst, jnp.zeros((pad,), dst.dtype)])
        adj = jnp.concatenate([adj_values, jnp.zeros((pad,), adj_values.dtype)])
    else:
        adj = adj_values
    src3 = src.reshape(NS, nb, EB)
    dst3 = dst.reshape(NS, nb, EB)
    val3 = adj.reshape(NS, nb, EB)
    zeros = jnp.zeros((n_nodes, dh), jnp.float32)

    x2 = jnp.concatenate([x[:, :dh], x[:, dh:]], axis=0)

    h2 = _sc_appnp(x2, src3, dst3, val3, zeros, n_nodes, dh, nb)

    return jnp.concatenate([h2[:n_nodes], h2[n_nodes:]], axis=1)
